# Initial kernel scaffold; baseline (speedup 1.0000x reference)
import jax
import jax.numpy as jnp
from jax.experimental import pallas as pl


def kernel(x, weight):
    raise NotImplementedError("write your pallas kernel here")


# speedup vs baseline: 1.1023x; 1.1023x over previous
import functools

import jax
import jax.numpy as jnp
from jax import lax
from jax.experimental import pallas as pl
from jax.experimental.pallas import tpu as pltpu
from jax.experimental.pallas import tpu_sc as plsc

NUM_EMB = 1000000
DIM = 32
ROWS = 16384
COLS = 50
B_TOTAL = ROWS * COLS
NW = 32
B_PER_W = B_TOTAL // NW
CHUNK = 1600
N_CHUNKS = B_PER_W // CHUNK


def _emb_body(idx_hbm, table_hbm, out_hbm, idx_v, rows_v, sem):
    wid = lax.axis_index("s") * 2 + lax.axis_index("c")
    base = wid * B_PER_W

    def chunk_step(i, carry):
        off = base + i * CHUNK
        pltpu.sync_copy(idx_hbm.at[pl.ds(off, CHUNK)], idx_v)
        pltpu.async_copy(table_hbm.at[idx_v], rows_v, sem).wait()
        pltpu.sync_copy(rows_v, out_hbm.at[pl.ds(off, CHUNK)])
        return carry

    lax.fori_loop(0, N_CHUNKS, chunk_step, 0)


@jax.jit
def _embedding_lookup(x_flat, weight):
    mesh = plsc.VectorSubcoreMesh(core_axis_name="c", subcore_axis_name="s")
    run = pl.kernel(
        _emb_body,
        mesh=mesh,
        out_type=jax.ShapeDtypeStruct((B_TOTAL, DIM), jnp.float32),
        scratch_types=[
            pltpu.VMEM((CHUNK,), jnp.int32),
            pltpu.VMEM((CHUNK, DIM), jnp.float32),
            pltpu.SemaphoreType.DMA,
        ],
        compiler_params=pltpu.CompilerParams(use_tc_tiling_on_sc=False),
    )
    return run(x_flat, weight)


def kernel(x, weight):
    x_flat = x.reshape((B_TOTAL,)).astype(jnp.int32)
    out = _embedding_lookup(x_flat, weight)
    return out.reshape((ROWS, COLS, DIM))


# speedup vs baseline: 1.1059x; 1.0033x over previous
import jax
import jax.numpy as jnp
from jax import lax
from jax.experimental import pallas as pl
from jax.experimental.pallas import tpu as pltpu
from jax.experimental.pallas import tpu_sc as plsc

NUM_EMB = 1000000
DIM = 32
ROWS = 16384
COLS = 50
B_TOTAL = ROWS * COLS
NW = 32
B_PER_W = B_TOTAL // NW
NBUF = 4
CHUNK = 800
N_CHUNKS = B_PER_W // CHUNK
N_OUTER = N_CHUNKS // NBUF


def _emb_body(idx_hbm, table_hbm, out_hbm, idx_v, rows_v, gsem, osem):
    wid = lax.axis_index("s") * 2 + lax.axis_index("c")
    base = wid * B_PER_W

    def load_idx(c, b):
        pltpu.sync_copy(idx_hbm.at[pl.ds(base + c * CHUNK, CHUNK)], idx_v.at[b])

    def start_gather(b):
        pltpu.async_copy(table_hbm.at[idx_v.at[b]], rows_v.at[b], gsem.at[b])

    for b in range(NBUF):
        load_idx(b, b)
        start_gather(b)

    def outer_step(o, carry):
        for b in range(NBUF):
            c = o * NBUF + b
            pltpu.make_async_copy(table_hbm.at[idx_v.at[b]], rows_v.at[b],
                                  gsem.at[b]).wait()
            pltpu.async_copy(rows_v.at[b],
                             out_hbm.at[pl.ds(base + c * CHUNK, CHUNK)],
                             osem.at[b])

            @pl.when(c + NBUF < N_CHUNKS)
            def _():
                pltpu.make_async_copy(
                    rows_v.at[b],
                    out_hbm.at[pl.ds(base + c * CHUNK, CHUNK)],
                    osem.at[b]).wait()
                load_idx(c + NBUF, b)
                start_gather(b)
        return carry

    lax.fori_loop(0, N_OUTER, outer_step, 0)

    for b in range(NBUF):
        c = N_CHUNKS - NBUF + b
        pltpu.make_async_copy(rows_v.at[b],
                              out_hbm.at[pl.ds(base + c * CHUNK, CHUNK)],
                              osem.at[b]).wait()


@jax.jit
def _embedding_lookup(x_flat, weight):
    mesh = plsc.VectorSubcoreMesh(core_axis_name="c", subcore_axis_name="s")
    run = pl.kernel(
        _emb_body,
        mesh=mesh,
        out_type=jax.ShapeDtypeStruct((B_TOTAL, DIM), jnp.float32),
        scratch_types=[
            pltpu.VMEM((NBUF, CHUNK), jnp.int32),
            pltpu.VMEM((NBUF, CHUNK, DIM), jnp.float32),
            pltpu.SemaphoreType.DMA((NBUF,)),
            pltpu.SemaphoreType.DMA((NBUF,)),
        ],
        compiler_params=pltpu.CompilerParams(use_tc_tiling_on_sc=False),
    )
    return run(x_flat, weight)


def kernel(x, weight):
    x_flat = x.reshape((B_TOTAL,)).astype(jnp.int32)
    out = _embedding_lookup(x_flat, weight)
    return out.reshape((ROWS, COLS, DIM))


# speedup vs baseline: 1.7908x; 1.6193x over previous
import jax
import jax.numpy as jnp
from jax import lax
from jax.experimental import pallas as pl
from jax.experimental.pallas import tpu as pltpu
from jax.experimental.pallas import tpu_sc as plsc

NUM_EMB = 1000000
DIM = 32
ROWS = 16384
COLS = 50
NW = 32
RB = 16
CHUNK = RB * COLS
NBUF = 4
N_BLOCKS = ROWS // RB
BLK_PER_W = N_BLOCKS // NW
N_OUTER = BLK_PER_W // NBUF


def _emb_body(x_hbm, table_hbm, out_hbm, idx_v, rows_v, gsem, osem):
    wid = lax.axis_index("s") * 2 + lax.axis_index("c")
    base = wid * BLK_PER_W

    def load_idx(c, b):
        pltpu.sync_copy(x_hbm.at[base + c], idx_v.at[b])

    def start_gather(b):
        pltpu.async_copy(table_hbm.at[idx_v.at[b]], rows_v.at[b], gsem.at[b])

    def wait_gather(b):
        pltpu.make_async_copy(table_hbm.at[idx_v.at[b]], rows_v.at[b],
                              gsem.at[b]).wait()

    def start_stores(c, b):
        r0 = (base + c) * RB
        for k in range(RB):
            pltpu.async_copy(rows_v.at[b, pl.ds(k * COLS, COLS)],
                             out_hbm.at[r0 + k], osem.at[b])

    def wait_stores(c, b):
        r0 = (base + c) * RB
        for k in range(RB):
            pltpu.make_async_copy(rows_v.at[b, pl.ds(k * COLS, COLS)],
                                  out_hbm.at[r0 + k], osem.at[b]).wait()

    for b in range(NBUF):
        load_idx(b, b)
        start_gather(b)

    def outer_step(o, carry):
        for b in range(NBUF):
            c = o * NBUF + b
            wait_gather(b)
            start_stores(c, b)

            @pl.when(c + NBUF < BLK_PER_W)
            def _():
                wait_stores(c, b)
                load_idx(c + NBUF, b)
                start_gather(b)
        return carry

    lax.fori_loop(0, N_OUTER, outer_step, 0)

    for b in range(NBUF):
        wait_stores(BLK_PER_W - NBUF + b, b)


@jax.jit
def _embedding_lookup(x_blocks, weight):
    mesh = plsc.VectorSubcoreMesh(core_axis_name="c", subcore_axis_name="s")
    run = pl.kernel(
        _emb_body,
        mesh=mesh,
        out_type=jax.ShapeDtypeStruct((ROWS, COLS, DIM), jnp.float32),
        scratch_types=[
            pltpu.VMEM((NBUF, CHUNK), jnp.int32),
            pltpu.VMEM((NBUF, CHUNK, DIM), jnp.float32),
            pltpu.SemaphoreType.DMA((NBUF,)),
            pltpu.SemaphoreType.DMA((NBUF,)),
        ],
        compiler_params=pltpu.CompilerParams(use_tc_tiling_on_sc=False),
    )
    return run(x_blocks, weight)


def kernel(x, weight):
    x_blocks = x.reshape((N_BLOCKS, CHUNK)).astype(jnp.int32)
    return _embedding_lookup(x_blocks, weight)
